# baseline (speedup 1.0000x reference)
import functools

import jax
import jax.numpy as jnp
from jax import lax
from jax.experimental import pallas as pl
from jax.experimental.pallas import tpu as pltpu
from jax.experimental.pallas import tpu_sc as plsc

K = 8192
D = 256
N = 8192
BETA = 0.25
BN = 256
NSTEPS = N // BN

NW = 32
B_PER_W = N // NW


CHUNK = 2048
NCHUNK = K // CHUNK


def _dist_argmin_body(zb_ref, zn_ref, wn_ref, wtb_ref, idx_ref, loss_ref):
    i = pl.program_id(0)

    mm2 = jnp.dot(zb_ref[...], wtb_ref[...],
                  preferred_element_type=jnp.float32)
    t1 = zn_ref[...] + wn_ref[...]
    d = t1 + mm2

    iota = lax.broadcasted_iota(jnp.int32, (BN, CHUNK), 1)

    def chunk_amin(seg):
        m = jnp.min(seg, axis=1)
        ic = jnp.min(jnp.where(seg == m[:, None], iota, K), axis=1)
        return m, ic.astype(jnp.int32)

    m, idx = chunk_amin(d[:, 0:CHUNK])
    best = m.astype(jnp.bfloat16).astype(jnp.float32)
    dpick = m
    for c in range(1, NCHUNK):
        m, ic = chunk_amin(d[:, c * CHUNK:(c + 1) * CHUNK])
        ic = ic + c * CHUNK
        upd = m < best
        best = jnp.where(upd, m.astype(jnp.bfloat16).astype(jnp.float32), best)
        idx = jnp.where(upd, ic, idx)
        dpick = jnp.where(upd, m, dpick)

    idx_ref[0, 0, :] = idx
    s = jnp.sum(dpick)

    @pl.when(i == 0)
    def _():
        loss_ref[0, 0] = s

    @pl.when(i > 0)
    def _():
        loss_ref[0, 0] += s

    @pl.when(i == NSTEPS - 1)
    def _():
        loss_ref[0, 0] *= BETA / (N * D)


def _dist_argmin(zb, z_norm, w_norm, wtb):
    return pl.pallas_call(
        _dist_argmin_body,
        grid=(NSTEPS,),
        in_specs=[
            pl.BlockSpec((BN, D), lambda i: (i, 0)),
            pl.BlockSpec((BN, 1), lambda i: (i, 0)),
            pl.BlockSpec((1, K), lambda i: (0, 0)),
            pl.BlockSpec((D, K), lambda i: (0, 0)),
        ],
        out_specs=[
            pl.BlockSpec((1, 1, BN), lambda i: (i, 0, 0)),
            pl.BlockSpec((1, 1), lambda i: (0, 0), memory_space=pltpu.SMEM),
        ],
        out_shape=[
            jax.ShapeDtypeStruct((NSTEPS, 1, BN), jnp.int32),
            jax.ShapeDtypeStruct((1, 1), jnp.float32),
        ],
    )(zb, z_norm, w_norm, wtb)


@functools.cache
def _make_sc_gather():
    mesh = plsc.VectorSubcoreMesh(core_axis_name="c", subcore_axis_name="s")

    @functools.partial(
        pl.kernel,
        mesh=mesh,
        out_type=jax.ShapeDtypeStruct((N, D), jnp.float32),
        scratch_types=[
            pltpu.VMEM((B_PER_W,), jnp.int32),
            pltpu.VMEM((B_PER_W, D), jnp.float32),
            pltpu.SemaphoreType.DMA,
        ],
    )
    def sc_gather(table_hbm, idx_hbm, out_hbm, idx_v, rows_v, sem):
        wid = lax.axis_index("s") * 2 + lax.axis_index("c")
        base = wid * B_PER_W
        pltpu.sync_copy(idx_hbm.at[pl.ds(base, B_PER_W)], idx_v)
        pltpu.async_copy(table_hbm.at[idx_v], rows_v, sem).wait()
        pltpu.sync_copy(rows_v, out_hbm.at[pl.ds(base, B_PER_W)])

    return sc_gather


def _st_body(z_ref, zq_ref, out_ref):
    z = z_ref[...]
    out_ref[...] = z + (zq_ref[...] - z)


def _st(z_flat, z_q):
    return pl.pallas_call(
        _st_body,
        grid=(8,),
        in_specs=[
            pl.BlockSpec((N // 8, D), lambda i: (i, 0)),
            pl.BlockSpec((N // 8, D), lambda i: (i, 0)),
        ],
        out_specs=pl.BlockSpec((N // 8, D), lambda i: (i, 0)),
        out_shape=jax.ShapeDtypeStruct((N, D), jnp.float32),
    )(z_flat, z_q)


def kernel(z_e, W):
    z_flat = z_e.reshape(N, D)
    zb = (z_flat * -2.0).astype(jnp.bfloat16)
    wtb = W.T.astype(jnp.bfloat16)
    z_norm = jnp.sum(z_flat ** 2, axis=1, keepdims=True)
    w_norm = jnp.sum(W ** 2, axis=1).reshape(1, K)

    idx3, loss_acc = _dist_argmin(zb, z_norm, w_norm, wtb)
    idx_flat = idx3.reshape(N)

    z_q = _make_sc_gather()(W, idx_flat)
    z_q_st = _st(z_flat, z_q).reshape(z_e.shape)

    indices = idx_flat.reshape(z_e.shape[:-1])
    loss = loss_acc[0, 0]
    return (z_q_st, indices, loss)


# speedup vs baseline: 1.1528x; 1.1528x over previous
import functools

import jax
import jax.numpy as jnp
from jax import lax
from jax.experimental import pallas as pl
from jax.experimental.pallas import tpu as pltpu
from jax.experimental.pallas import tpu_sc as plsc

K = 8192
D = 256
N = 8192
BETA = 0.25
BN = 256
NSTEPS = N // BN

NW = 32
B_PER_W = N // NW


CHUNK = 2048
NCHUNK = K // CHUNK


SUBB = 64


def _dist_argmin_body(z_ref, zn_ref, wn_ref, wtb_ref, idx_ref, loss_ref):
    i = pl.program_id(0)

    zb = (z_ref[...] * -2.0).astype(jnp.bfloat16)
    mm2 = jnp.dot(zb, wtb_ref[...],
                  preferred_element_type=jnp.float32)
    t1 = zn_ref[...] + wn_ref[...]
    d = t1 + mm2

    lane = lax.broadcasted_iota(jnp.int32, (SUBB, 128), 1)

    parts = []
    for rb in range(BN // SUBB):
        dsub = d[rb * SUBB:(rb + 1) * SUBB, :]
        idx = None
        for c in range(NCHUNK):
            c_hi = (c + 1) * (CHUNK // 128) - 1
            best_l = dsub[:, c_hi * 128:(c_hi + 1) * 128]
            bcol = jnp.full((SUBB, 128), c_hi, jnp.int32)
            for cc in range(c_hi - 1, c * (CHUNK // 128) - 1, -1):
                v = dsub[:, cc * 128:(cc + 1) * 128]
                upd = v <= best_l
                best_l = jnp.where(upd, v, best_l)
                bcol = jnp.where(upd, cc, bcol)
            m = jnp.min(best_l, axis=1)
            kk = bcol * 128 + lane
            cand = jnp.where(best_l == m[:, None], kk, K)
            ic = jnp.min(cand, axis=1).astype(jnp.int32)
            if idx is None:
                best = m.astype(jnp.bfloat16).astype(jnp.float32)
                idx = ic
                dpick = m
            else:
                upd = m < best
                best = jnp.where(upd,
                                 m.astype(jnp.bfloat16).astype(jnp.float32),
                                 best)
                idx = jnp.where(upd, ic, idx)
                dpick = jnp.where(upd, m, dpick)
        idx_ref[0, 0, rb * SUBB:(rb + 1) * SUBB] = idx
        parts.append(jnp.sum(dpick))

    s = sum(parts)

    @pl.when(i == 0)
    def _():
        loss_ref[0, 0] = s

    @pl.when(i > 0)
    def _():
        loss_ref[0, 0] += s

    @pl.when(i == NSTEPS - 1)
    def _():
        loss_ref[0, 0] *= BETA / (N * D)


def _dist_argmin(zb, z_norm, w_norm, wtb):
    return pl.pallas_call(
        _dist_argmin_body,
        grid=(NSTEPS,),
        in_specs=[
            pl.BlockSpec((BN, D), lambda i: (i, 0)),
            pl.BlockSpec((BN, 1), lambda i: (i, 0)),
            pl.BlockSpec((1, K), lambda i: (0, 0)),
            pl.BlockSpec((D, K), lambda i: (0, 0)),
        ],
        out_specs=[
            pl.BlockSpec((1, 1, BN), lambda i: (i, 0, 0)),
            pl.BlockSpec((1, 1), lambda i: (0, 0), memory_space=pltpu.SMEM),
        ],
        out_shape=[
            jax.ShapeDtypeStruct((NSTEPS, 1, BN), jnp.int32),
            jax.ShapeDtypeStruct((1, 1), jnp.float32),
        ],
    )(zb, z_norm, w_norm, wtb)


@functools.cache
def _make_sc_gather():
    mesh = plsc.VectorSubcoreMesh(core_axis_name="c", subcore_axis_name="s")

    @functools.partial(
        pl.kernel,
        mesh=mesh,
        out_type=jax.ShapeDtypeStruct((N, D), jnp.float32),
        scratch_types=[
            pltpu.VMEM((B_PER_W,), jnp.int32),
            pltpu.VMEM((B_PER_W, D), jnp.float32),
            pltpu.SemaphoreType.DMA,
        ],
    )
    def sc_gather(table_hbm, idx_hbm, out_hbm, idx_v, rows_v, sem):
        wid = lax.axis_index("s") * 2 + lax.axis_index("c")
        base = wid * B_PER_W
        pltpu.sync_copy(idx_hbm.at[pl.ds(base, B_PER_W)], idx_v)
        pltpu.async_copy(table_hbm.at[idx_v], rows_v, sem).wait()
        pltpu.sync_copy(rows_v, out_hbm.at[pl.ds(base, B_PER_W)])

    return sc_gather


def _st_body(z_ref, zq_ref, out_ref):
    z = z_ref[...]
    out_ref[...] = z + (zq_ref[...] - z)


def _st(z_flat, z_q):
    return pl.pallas_call(
        _st_body,
        grid=(8,),
        in_specs=[
            pl.BlockSpec((N // 8, D), lambda i: (i, 0)),
            pl.BlockSpec((N // 8, D), lambda i: (i, 0)),
        ],
        out_specs=pl.BlockSpec((N // 8, D), lambda i: (i, 0)),
        out_shape=jax.ShapeDtypeStruct((N, D), jnp.float32),
    )(z_flat, z_q)


def kernel(z_e, W):
    z_flat = z_e.reshape(N, D)
    wtb = W.T.astype(jnp.bfloat16)
    z_norm = jnp.sum(z_flat ** 2, axis=1, keepdims=True)
    w_norm = jnp.sum(W ** 2, axis=1).reshape(1, K)

    idx3, loss_acc = _dist_argmin(z_flat, z_norm, w_norm, wtb)
    idx_flat = idx3.reshape(N)

    z_q = _make_sc_gather()(W, idx_flat)
    z_q_st = _st(z_flat, z_q).reshape(z_e.shape)

    indices = idx_flat.reshape(z_e.shape[:-1])
    loss = loss_acc[0, 0]
    return (z_q_st, indices, loss)
